# baseline (speedup 1.0000x reference)
import jax
import jax.numpy as jnp
from jax import lax
from jax.experimental import pallas as pl
from jax.experimental.pallas import tpu as pltpu
from jax.experimental.pallas import tpu_sc as plsc

_N = 10000
_E = 320000
_D = 128
_G = 64

_NC = 2
_NS = 16
_NT = _NC * _NS
_K = 64
_NBLK = 160
_P = 4
_TOTBLK = _NT * _NBLK
_FF = 288
_FS = 2 * _NBLK - _FF
_EPAD = _NT * _NBLK * _K
_NPAD = 10112
_STRIPE = _NPAD // _NS
_CH = 16
_NCHUNK = _NBLK // _CH
_ZB = 640

_f32 = jnp.float32


def _sc_deg_body(cols_hbm, out0_hbm, out1_hbm, cbuf, ones_v, zbuf, acc):
    c = lax.axis_index("c")
    s = lax.axis_index("s")
    w = s * _NC + c
    wbase = pl.multiple_of(w * _NBLK, 16)
    pltpu.sync_copy(cols_hbm.at[pl.ds(wbase, _NBLK)], cbuf)
    ones16 = jnp.ones((16,), _f32)
    zero16 = jnp.zeros((16,), _f32)
    for i in range(_K // 16):
        ones_v[pl.ds(i * 16, 16)] = ones16
    for i in range(_ZB // 16):
        zbuf[pl.ds(i * 16, 16)] = zero16
    base = pl.multiple_of(s * _STRIPE, 8)
    pltpu.sync_copy(zbuf.at[pl.ds(0, _STRIPE)], acc.at[pl.ds(base, _STRIPE)])
    plsc.subcore_barrier()

    def body(j, carry):
        pltpu.sync_copy(ones_v, acc.at[cbuf.at[j]], add=True)
        return carry

    lax.fori_loop(0, _NBLK, body, 0)
    plsc.subcore_barrier()

    @pl.when(jnp.logical_and(c == 0, s == 0))
    def _():
        pltpu.sync_copy(acc, out0_hbm)

    @pl.when(jnp.logical_and(c == 1, s == 0))
    def _():
        pltpu.sync_copy(acc, out1_hbm)


def _sc_agg_body(h_hbm, rows_hbm, cols_hbm, out_hbm, rbuf, cbuf,
                 g0, g1, g2, g3, acc, sem0, sem1, sem2, sem3):
    c = lax.axis_index("c")
    s = lax.axis_index("s")
    myblk = pl.multiple_of(jnp.where(c == 1, s * _FF, 16 * _FF + s * _FS), 16)
    nchunk = jnp.where(c == 1, _FF // _CH, _FS // _CH)
    bufs = [g0, g1, g2, g3]
    sems = [sem0, sem1, sem2, sem3]
    zero16 = jnp.zeros((16,), _f32)

    def zrow(r, carry):
        for i in range(_D // 16):
            g0[r, pl.ds(i * 16, 16)] = zero16
        return carry

    lax.fori_loop(0, _K, zrow, 0)
    base = pl.multiple_of(s * _STRIPE, 8)
    nfull, rem = divmod(_STRIPE, _K)
    for i in range(nfull):
        pltpu.sync_copy(g0, acc.at[pl.ds(base + i * _K, _K)])
    if rem:
        pltpu.sync_copy(g0.at[pl.ds(0, rem)],
                        acc.at[pl.ds(base + nfull * _K, rem)])
    plsc.subcore_barrier()

    def chunk_body(ci, carry):
        cbase = pl.multiple_of(myblk + ci * _CH, 16)
        pltpu.sync_copy(rows_hbm.at[pl.ds(cbase, _CH)], rbuf)
        pltpu.sync_copy(cols_hbm.at[pl.ds(cbase, _CH)], cbuf)
        for b in range(_P):
            pltpu.async_copy(h_hbm.at[rbuf.at[b]], bufs[b], sems[b])
        for b in range(_CH):
            gb = bufs[b % _P]
            pltpu.make_async_copy(h_hbm.at[rbuf.at[b]], gb,
                                  sems[b % _P]).wait()
            pltpu.sync_copy(gb, acc.at[cbuf.at[b]], add=True)
            if b + _P < _CH:
                pltpu.async_copy(h_hbm.at[rbuf.at[b + _P]], gb, sems[b % _P])
        return carry

    lax.fori_loop(0, nchunk, chunk_body, 0)
    plsc.subcore_barrier()
    for i in range(nfull):
        pltpu.sync_copy(acc.at[pl.ds(base + i * _K, _K)],
                        out_hbm.at[c, pl.ds(base + i * _K, _K)])
    if rem:
        pltpu.sync_copy(acc.at[pl.ds(base + nfull * _K, rem)],
                        out_hbm.at[c, pl.ds(base + nfull * _K, rem)])


def _make_sc_calls():
    mesh = plsc.VectorSubcoreMesh(core_axis_name="c", subcore_axis_name="s",
                                  num_cores=_NC)
    deg_call = pl.kernel(
        _sc_deg_body,
        mesh=mesh,
        out_type=[jax.ShapeDtypeStruct((_NPAD,), _f32),
                  jax.ShapeDtypeStruct((_NPAD,), _f32)],
        scratch_types=[
            pltpu.VMEM((_NBLK, _K), jnp.int32),
            pltpu.VMEM((_K,), _f32),
            pltpu.VMEM((_ZB,), _f32),
            pltpu.VMEM_SHARED((_NPAD,), _f32),
        ],
    )
    agg_call = pl.kernel(
        _sc_agg_body,
        mesh=mesh,
        out_type=jax.ShapeDtypeStruct((_NC, _NPAD, _D), _f32),
        scratch_types=[
            pltpu.VMEM((_CH, _K), jnp.int32),
            pltpu.VMEM((_CH, _K), jnp.int32),
            pltpu.VMEM((_K, _D), _f32),
            pltpu.VMEM((_K, _D), _f32),
            pltpu.VMEM((_K, _D), _f32),
            pltpu.VMEM((_K, _D), _f32),
            pltpu.VMEM_SHARED((_NPAD, _D), _f32),
            pltpu.SemaphoreType.DMA,
            pltpu.SemaphoreType.DMA,
            pltpu.SemaphoreType.DMA,
            pltpu.SemaphoreType.DMA,
        ],
    )
    return deg_call, agg_call


_bf16 = jnp.bfloat16
_DN = (((0,), (0,)), ((), ()))
_DG = (((1,), (0,)), ((), ()))


def _bsplit(y):
    yh = y.astype(_bf16)
    yl = (y - yh.astype(_f32)).astype(_bf16)
    return yh, yl


def _dot_exact_split(a_exact, y, dims):
    yh, yl = _bsplit(y)
    return (lax.dot_general(a_exact, yh, dims, preferred_element_type=_f32) +
            lax.dot_general(a_exact, yl, dims, preferred_element_type=_f32))


def _dot_split_split(g, w, dims):
    gh, gl = _bsplit(g)
    wh, wl = _bsplit(w)
    return (lax.dot_general(gh, wh, dims, preferred_element_type=_f32) +
            lax.dot_general(gl, wh, dims, preferred_element_type=_f32) +
            lax.dot_general(gh, wl, dims, preferred_element_type=_f32) +
            lax.dot_general(gl, wl, dims, preferred_element_type=_f32))


def _onehot_cnt(bcol):
    iog = lax.broadcasted_iota(jnp.int32, (_N, _G), 1)
    ohb = (bcol == iog).astype(_bf16)
    ones_n = jnp.ones((_N, 1), _bf16)
    cnt = jnp.maximum(
        lax.dot_general(ohb, ones_n, _DN, preferred_element_type=_f32), 1.0)
    return ohb, cnt


def _graph_norm(xv, ohb, cnt, w, b, ms):
    mean = _dot_exact_split(ohb, xv, _DN) / cnt
    meanb = _dot_exact_split(ohb, mean, _DG)
    out = xv - meanb * ms
    var = _dot_exact_split(ohb, out * out, _DN) / cnt
    istd = lax.rsqrt(var + 1e-5)
    istdb = _dot_exact_split(ohb, istd, _DG)
    return w * out * istdb + b


def _tc1_body(x_ref, b_ref, d0_ref, d1_ref, gw_ref, gb_ref, gms_ref, w1_ref,
              h1p_ref, dinv_ref):
    dinv = lax.rsqrt(d0_ref[...] + d1_ref[...] + 1.0)
    ohb, cnt = _onehot_cnt(b_ref[...])
    g = _graph_norm(x_ref[...], ohb, cnt, gw_ref[...], gb_ref[...], gms_ref[...])
    h1 = _dot_split_split(g, w1_ref[...], _DG)
    h1p_ref[...] = h1 * dinv
    dinv_ref[...] = dinv


def _tc2_body(acc_ref, h1p_ref, dinv_ref, b_ref, b1_ref, gw_ref, gb_ref,
              gms_ref, w2_ref, h2p_ref):
    dinv = dinv_ref[...]
    agg = acc_ref[0, : _N, :] + acc_ref[1, : _N, :]
    a1 = dinv * (agg + h1p_ref[...]) + b1_ref[...]
    r = jnp.maximum(a1, 0.0)
    ohb, cnt = _onehot_cnt(b_ref[...])
    g = _graph_norm(r, ohb, cnt, gw_ref[...], gb_ref[...], gms_ref[...])
    h2 = _dot_split_split(g, w2_ref[...], _DG)
    h2p_ref[...] = h2 * dinv


def _tc3_body(acc_ref, h2p_ref, dinv_ref, b_ref, b2_ref, out_ref):
    agg = acc_ref[0, : _N, :] + acc_ref[1, : _N, :]
    a2 = dinv_ref[...] * (agg + h2p_ref[...]) + b2_ref[...]
    ohb, cnt = _onehot_cnt(b_ref[...])
    out_ref[...] = _dot_exact_split(ohb, a2, _DN) / cnt


_tc_params = pltpu.CompilerParams(vmem_limit_bytes=120 * 1024 * 1024)

_tc1_call = pl.pallas_call(
    _tc1_body,
    out_shape=[jax.ShapeDtypeStruct((_N, _D), _f32),
               jax.ShapeDtypeStruct((_N, 1), _f32)],
    compiler_params=_tc_params)

_tc2_call = pl.pallas_call(
    _tc2_body,
    out_shape=jax.ShapeDtypeStruct((_N, _D), _f32),
    compiler_params=_tc_params)

_tc3_call = pl.pallas_call(
    _tc3_body,
    out_shape=jax.ShapeDtypeStruct((_G, _D), _f32),
    compiler_params=_tc_params)

_sc_calls_cache = []


def _get_sc_calls():
    if not _sc_calls_cache:
        _sc_calls_cache.append(_make_sc_calls())
    return _sc_calls_cache[0]


def kernel(x, edge_index, batch, gn1_w, gn1_b, gn1_ms, W1, b1,
           gn2_w, gn2_b, gn2_ms, W2, b2):
    bcol = batch.reshape(_N, 1)
    gw1 = gn1_w.reshape(1, _D)
    gb1 = gn1_b.reshape(1, _D)
    gms1 = gn1_ms.reshape(1, _D)
    gw2 = gn2_w.reshape(1, _D)
    gb2 = gn2_b.reshape(1, _D)
    gms2 = gn2_ms.reshape(1, _D)
    b1r = b1.reshape(1, _D)
    b2r = b2.reshape(1, _D)

    pad = _EPAD - _E
    rows_p = jnp.concatenate(
        [edge_index[0], jnp.zeros((pad,), edge_index.dtype)]
    ).reshape(_TOTBLK, _K)
    cols_p = jnp.concatenate(
        [edge_index[1], jnp.full((pad,), _N, edge_index.dtype)]
    ).reshape(_TOTBLK, _K)

    _deg_call, _agg_call = _get_sc_calls()
    degp0, degp1 = _deg_call(cols_p)
    d0 = degp0[:_N].reshape(_N, 1)
    d1 = degp1[:_N].reshape(_N, 1)

    h1p, dinv = _tc1_call(x, bcol, d0, d1, gw1, gb1, gms1, W1)
    acc1 = _agg_call(h1p, rows_p, cols_p)
    h2p = _tc2_call(acc1, h1p, dinv, bcol, b1r, gw2, gb2, gms2, W2)
    acc2 = _agg_call(h2p, rows_p, cols_p)
    return _tc3_call(acc2, h2p, dinv, bcol, b2r)


# speedup vs baseline: 1.0042x; 1.0042x over previous
import jax
import jax.numpy as jnp
from jax import lax
from jax.experimental import pallas as pl
from jax.experimental.pallas import tpu as pltpu
from jax.experimental.pallas import tpu_sc as plsc

_N = 10000
_E = 320000
_D = 128
_G = 64

_NC = 2
_NS = 16
_NT = _NC * _NS
_K = 64
_NBLK = 160
_P = 4
_TOTBLK = _NT * _NBLK
_FF = 256
_FS = 2 * _NBLK - _FF
_EPAD = _NT * _NBLK * _K
_NPAD = 10112
_STRIPE = _NPAD // _NS
_CH = 16
_NCHUNK = _NBLK // _CH
_ZB = 640

_f32 = jnp.float32


def _sc_deg_body(cols_hbm, out0_hbm, out1_hbm, cbuf, ones_v, zbuf, acc):
    c = lax.axis_index("c")
    s = lax.axis_index("s")
    w = s * _NC + c
    wbase = pl.multiple_of(w * _NBLK, 16)
    pltpu.sync_copy(cols_hbm.at[pl.ds(wbase, _NBLK)], cbuf)
    ones16 = jnp.ones((16,), _f32)
    zero16 = jnp.zeros((16,), _f32)
    for i in range(_K // 16):
        ones_v[pl.ds(i * 16, 16)] = ones16
    for i in range(_ZB // 16):
        zbuf[pl.ds(i * 16, 16)] = zero16
    base = pl.multiple_of(s * _STRIPE, 8)
    pltpu.sync_copy(zbuf.at[pl.ds(0, _STRIPE)], acc.at[pl.ds(base, _STRIPE)])
    plsc.subcore_barrier()

    def body(j, carry):
        pltpu.sync_copy(ones_v, acc.at[cbuf.at[j]], add=True)
        return carry

    lax.fori_loop(0, _NBLK, body, 0)
    plsc.subcore_barrier()

    @pl.when(jnp.logical_and(c == 0, s == 0))
    def _():
        pltpu.sync_copy(acc, out0_hbm)

    @pl.when(jnp.logical_and(c == 1, s == 0))
    def _():
        pltpu.sync_copy(acc, out1_hbm)


def _sc_agg_body(h_hbm, rows_hbm, cols_hbm, out_hbm, rbuf, cbuf,
                 g0, g1, g2, g3, acc, sem0, sem1, sem2, sem3):
    c = lax.axis_index("c")
    s = lax.axis_index("s")
    myblk = pl.multiple_of(jnp.where(c == 1, s * _FF, 16 * _FF + s * _FS), 16)
    nchunk = jnp.where(c == 1, _FF // _CH, _FS // _CH)
    bufs = [g0, g1, g2, g3]
    sems = [sem0, sem1, sem2, sem3]
    zero16 = jnp.zeros((16,), _f32)

    def zrow(r, carry):
        for i in range(_D // 16):
            g0[r, pl.ds(i * 16, 16)] = zero16
        return carry

    lax.fori_loop(0, _K, zrow, 0)
    base = pl.multiple_of(s * _STRIPE, 8)
    nfull, rem = divmod(_STRIPE, _K)
    for i in range(nfull):
        pltpu.sync_copy(g0, acc.at[pl.ds(base + i * _K, _K)])
    if rem:
        pltpu.sync_copy(g0.at[pl.ds(0, rem)],
                        acc.at[pl.ds(base + nfull * _K, rem)])
    plsc.subcore_barrier()

    def chunk_body(ci, carry):
        cbase = pl.multiple_of(myblk + ci * _CH, 16)
        pltpu.sync_copy(rows_hbm.at[pl.ds(cbase, _CH)], rbuf)
        pltpu.sync_copy(cols_hbm.at[pl.ds(cbase, _CH)], cbuf)
        for b in range(_P):
            pltpu.async_copy(h_hbm.at[rbuf.at[b]], bufs[b], sems[b])
        for b in range(_CH):
            gb = bufs[b % _P]
            pltpu.make_async_copy(h_hbm.at[rbuf.at[b]], gb,
                                  sems[b % _P]).wait()
            pltpu.sync_copy(gb, acc.at[cbuf.at[b]], add=True)
            if b + _P < _CH:
                pltpu.async_copy(h_hbm.at[rbuf.at[b + _P]], gb, sems[b % _P])
        return carry

    lax.fori_loop(0, nchunk, chunk_body, 0)
    plsc.subcore_barrier()
    for i in range(nfull):
        pltpu.sync_copy(acc.at[pl.ds(base + i * _K, _K)],
                        out_hbm.at[c, pl.ds(base + i * _K, _K)])
    if rem:
        pltpu.sync_copy(acc.at[pl.ds(base + nfull * _K, rem)],
                        out_hbm.at[c, pl.ds(base + nfull * _K, rem)])


def _make_sc_calls():
    mesh = plsc.VectorSubcoreMesh(core_axis_name="c", subcore_axis_name="s",
                                  num_cores=_NC)
    deg_call = pl.kernel(
        _sc_deg_body,
        mesh=mesh,
        out_type=[jax.ShapeDtypeStruct((_NPAD,), _f32),
                  jax.ShapeDtypeStruct((_NPAD,), _f32)],
        scratch_types=[
            pltpu.VMEM((_NBLK, _K), jnp.int32),
            pltpu.VMEM((_K,), _f32),
            pltpu.VMEM((_ZB,), _f32),
            pltpu.VMEM_SHARED((_NPAD,), _f32),
        ],
    )
    agg_call = pl.kernel(
        _sc_agg_body,
        mesh=mesh,
        out_type=jax.ShapeDtypeStruct((_NC, _NPAD, _D), _f32),
        scratch_types=[
            pltpu.VMEM((_CH, _K), jnp.int32),
            pltpu.VMEM((_CH, _K), jnp.int32),
            pltpu.VMEM((_K, _D), _f32),
            pltpu.VMEM((_K, _D), _f32),
            pltpu.VMEM((_K, _D), _f32),
            pltpu.VMEM((_K, _D), _f32),
            pltpu.VMEM_SHARED((_NPAD, _D), _f32),
            pltpu.SemaphoreType.DMA,
            pltpu.SemaphoreType.DMA,
            pltpu.SemaphoreType.DMA,
            pltpu.SemaphoreType.DMA,
        ],
    )
    return deg_call, agg_call


_bf16 = jnp.bfloat16
_DN = (((0,), (0,)), ((), ()))
_DG = (((1,), (0,)), ((), ()))


def _bsplit(y):
    yh = y.astype(_bf16)
    yl = (y - yh.astype(_f32)).astype(_bf16)
    return yh, yl


def _dot_exact_split(a_exact, y, dims):
    yh, yl = _bsplit(y)
    return (lax.dot_general(a_exact, yh, dims, preferred_element_type=_f32) +
            lax.dot_general(a_exact, yl, dims, preferred_element_type=_f32))


def _dot_split_split(g, w, dims):
    gh, gl = _bsplit(g)
    wh, wl = _bsplit(w)
    return (lax.dot_general(gh, wh, dims, preferred_element_type=_f32) +
            lax.dot_general(gl, wh, dims, preferred_element_type=_f32) +
            lax.dot_general(gh, wl, dims, preferred_element_type=_f32) +
            lax.dot_general(gl, wl, dims, preferred_element_type=_f32))


def _onehot_cnt(bcol):
    iog = lax.broadcasted_iota(jnp.int32, (_N, _G), 1)
    ohb = (bcol == iog).astype(_bf16)
    ones_n = jnp.ones((_N, 1), _bf16)
    cnt = jnp.maximum(
        lax.dot_general(ohb, ones_n, _DN, preferred_element_type=_f32), 1.0)
    return ohb, cnt


def _graph_norm(xv, ohb, cnt, w, b, ms):
    mean = _dot_exact_split(ohb, xv, _DN) / cnt
    meanb = _dot_exact_split(ohb, mean, _DG)
    out = xv - meanb * ms
    var = _dot_exact_split(ohb, out * out, _DN) / cnt
    istd = lax.rsqrt(var + 1e-5)
    istdb = _dot_exact_split(ohb, istd, _DG)
    return w * out * istdb + b


def _tc1_body(x_ref, b_ref, d0_ref, d1_ref, gw_ref, gb_ref, gms_ref, w1_ref,
              h1p_ref, dinv_ref):
    dinv = lax.rsqrt(d0_ref[...] + d1_ref[...] + 1.0)
    ohb, cnt = _onehot_cnt(b_ref[...])
    g = _graph_norm(x_ref[...], ohb, cnt, gw_ref[...], gb_ref[...], gms_ref[...])
    h1 = _dot_split_split(g, w1_ref[...], _DG)
    h1p_ref[...] = h1 * dinv
    dinv_ref[...] = dinv


def _tc2_body(acc_ref, h1p_ref, dinv_ref, b_ref, b1_ref, gw_ref, gb_ref,
              gms_ref, w2_ref, h2p_ref):
    dinv = dinv_ref[...]
    agg = acc_ref[0, : _N, :] + acc_ref[1, : _N, :]
    a1 = dinv * (agg + h1p_ref[...]) + b1_ref[...]
    r = jnp.maximum(a1, 0.0)
    ohb, cnt = _onehot_cnt(b_ref[...])
    g = _graph_norm(r, ohb, cnt, gw_ref[...], gb_ref[...], gms_ref[...])
    h2 = _dot_split_split(g, w2_ref[...], _DG)
    h2p_ref[...] = h2 * dinv


def _tc3_body(acc_ref, h2p_ref, dinv_ref, b_ref, b2_ref, out_ref):
    agg = acc_ref[0, : _N, :] + acc_ref[1, : _N, :]
    a2 = dinv_ref[...] * (agg + h2p_ref[...]) + b2_ref[...]
    ohb, cnt = _onehot_cnt(b_ref[...])
    out_ref[...] = _dot_exact_split(ohb, a2, _DN) / cnt


_tc_params = pltpu.CompilerParams(vmem_limit_bytes=120 * 1024 * 1024)

_tc1_call = pl.pallas_call(
    _tc1_body,
    out_shape=[jax.ShapeDtypeStruct((_N, _D), _f32),
               jax.ShapeDtypeStruct((_N, 1), _f32)],
    compiler_params=_tc_params)

_tc2_call = pl.pallas_call(
    _tc2_body,
    out_shape=jax.ShapeDtypeStruct((_N, _D), _f32),
    compiler_params=_tc_params)

_tc3_call = pl.pallas_call(
    _tc3_body,
    out_shape=jax.ShapeDtypeStruct((_G, _D), _f32),
    compiler_params=_tc_params)

_sc_calls_cache = []


def _get_sc_calls():
    if not _sc_calls_cache:
        _sc_calls_cache.append(_make_sc_calls())
    return _sc_calls_cache[0]


def kernel(x, edge_index, batch, gn1_w, gn1_b, gn1_ms, W1, b1,
           gn2_w, gn2_b, gn2_ms, W2, b2):
    bcol = batch.reshape(_N, 1)
    gw1 = gn1_w.reshape(1, _D)
    gb1 = gn1_b.reshape(1, _D)
    gms1 = gn1_ms.reshape(1, _D)
    gw2 = gn2_w.reshape(1, _D)
    gb2 = gn2_b.reshape(1, _D)
    gms2 = gn2_ms.reshape(1, _D)
    b1r = b1.reshape(1, _D)
    b2r = b2.reshape(1, _D)

    pad = _EPAD - _E
    rows_p = jnp.concatenate(
        [edge_index[0], jnp.zeros((pad,), edge_index.dtype)]
    ).reshape(_TOTBLK, _K)
    cols_p = jnp.concatenate(
        [edge_index[1], jnp.full((pad,), _N, edge_index.dtype)]
    ).reshape(_TOTBLK, _K)

    _deg_call, _agg_call = _get_sc_calls()
    degp0, degp1 = _deg_call(cols_p)
    d0 = degp0[:_N].reshape(_N, 1)
    d1 = degp1[:_N].reshape(_N, 1)

    h1p, dinv = _tc1_call(x, bcol, d0, d1, gw1, gb1, gms1, W1)
    acc1 = _agg_call(h1p, rows_p, cols_p)
    h2p = _tc2_call(acc1, h1p, dinv, bcol, b1r, gw2, gb2, gms2, W2)
    acc2 = _agg_call(h2p, rows_p, cols_p)
    return _tc3_call(acc2, h2p, dinv, bcol, b2r)
